# Initial kernel scaffold; baseline (speedup 1.0000x reference)
import jax
import jax.numpy as jnp
from jax.experimental import pallas as pl


def kernel(x, tiles):
    raise NotImplementedError("write your pallas kernel here")


# speedup vs baseline: 4.1671x; 4.1671x over previous
import jax
import jax.numpy as jnp
from jax.experimental import pallas as pl

_N = 131072
_DIM = 2
_BINS = 15
_NUM_OUTPUTS = (_BINS + 1) ** _DIM
_BLK = 1024


def _tile_code_block(x_ref, tiles_ref, out_ref):
    xb = x_ref[...]
    tiles = tiles_ref[...]
    cnt = jnp.zeros(xb.shape, jnp.int32)
    for b in range(_BINS):
        cnt = cnt + (xb > tiles[b : b + 1, :]).astype(jnp.int32)
    code = (_BINS + 1) * cnt[:, 0:1] + cnt[:, 1:2]
    cols = jax.lax.broadcasted_iota(jnp.int32, (_BLK, _NUM_OUTPUTS), 1)
    out_ref[...] = (cols == code).astype(jnp.float32)


def kernel(x, tiles):
    grid = _N // _BLK
    return pl.pallas_call(
        _tile_code_block,
        grid=(grid,),
        in_specs=[
            pl.BlockSpec((_BLK, _DIM), lambda i: (i, 0)),
            pl.BlockSpec((_BINS, _DIM), lambda i: (0, 0)),
        ],
        out_specs=pl.BlockSpec((_BLK, _NUM_OUTPUTS), lambda i: (i, 0)),
        out_shape=jax.ShapeDtypeStruct((_N, _NUM_OUTPUTS), jnp.float32),
    )(x, tiles)


# speedup vs baseline: 9.7878x; 2.3488x over previous
import jax
import jax.numpy as jnp
from jax.experimental import pallas as pl

_N = 131072
_DIM = 2
_BINS = 15
_BP = _BINS + 1
_NUM_OUTPUTS = _BP * _BP
_BLK = 2048
_SUB = _BLK // 128


def _tile_code_block(x0_ref, x1_ref, tiles_ref, out_ref):
    x0 = x0_ref[...]
    x1 = x1_ref[...]
    cnt0 = jnp.zeros(x0.shape, jnp.int32)
    cnt1 = jnp.zeros(x1.shape, jnp.int32)
    for b in range(_BINS):
        cnt0 = cnt0 + (x0 > tiles_ref[b : b + 1, 0:1]).astype(jnp.int32)
        cnt1 = cnt1 + (x1 > tiles_ref[b : b + 1, 1:2]).astype(jnp.int32)
    code = _BP * cnt0 + cnt1
    cols = jax.lax.broadcasted_iota(jnp.int32, (_SUB, _NUM_OUTPUTS), 1)
    for k in range(128):
        col = code[:, k : k + 1]
        out_ref[_SUB * k : _SUB * (k + 1), :] = (cols == col).astype(
            jnp.float32
        )


def kernel(x, tiles):
    nb = _N // _BLK
    x0g = x[:, 0].reshape(nb, 128, _SUB).transpose(0, 2, 1).reshape(nb * _SUB, 128)
    x1g = x[:, 1].reshape(nb, 128, _SUB).transpose(0, 2, 1).reshape(nb * _SUB, 128)
    return pl.pallas_call(
        _tile_code_block,
        grid=(nb,),
        in_specs=[
            pl.BlockSpec((_SUB, 128), lambda i: (i, 0)),
            pl.BlockSpec((_SUB, 128), lambda i: (i, 0)),
            pl.BlockSpec((_BINS, _DIM), lambda i: (0, 0)),
        ],
        out_specs=pl.BlockSpec((_BLK, _NUM_OUTPUTS), lambda i: (i, 0)),
        out_shape=jax.ShapeDtypeStruct((_N, _NUM_OUTPUTS), jnp.float32),
    )(x0g, x1g, tiles)


# speedup vs baseline: 14.8309x; 1.5152x over previous
import jax
import jax.numpy as jnp
from jax.experimental import pallas as pl

_N = 131072
_DIM = 2
_BINS = 15
_BP = _BINS + 1
_NUM_OUTPUTS = _BP * _BP
_BLK = 8192
_SUB = _BLK // 128


def _tile_code_block(x0_ref, x1_ref, tiles_ref, out_ref):
    x0 = x0_ref[...]
    x1 = x1_ref[...]
    cnt0 = jnp.zeros(x0.shape, jnp.int32)
    cnt1 = jnp.zeros(x1.shape, jnp.int32)
    for b in range(_BINS):
        cnt0 = cnt0 + (x0 > tiles_ref[b : b + 1, 0:1]).astype(jnp.int32)
        cnt1 = cnt1 + (x1 > tiles_ref[b : b + 1, 1:2]).astype(jnp.int32)
    code = _BP * cnt0 + cnt1
    cols = jax.lax.broadcasted_iota(jnp.int32, (_SUB, _NUM_OUTPUTS), 1)
    for k in range(128):
        col = code[:, k : k + 1]
        out_ref[_SUB * k : _SUB * (k + 1), :] = (cols == col).astype(
            jnp.float32
        )


def kernel(x, tiles):
    nb = _N // _BLK
    x0g = x[:, 0].reshape(nb, 128, _SUB).transpose(0, 2, 1).reshape(nb * _SUB, 128)
    x1g = x[:, 1].reshape(nb, 128, _SUB).transpose(0, 2, 1).reshape(nb * _SUB, 128)
    return pl.pallas_call(
        _tile_code_block,
        grid=(nb,),
        in_specs=[
            pl.BlockSpec((_SUB, 128), lambda i: (i, 0)),
            pl.BlockSpec((_SUB, 128), lambda i: (i, 0)),
            pl.BlockSpec((_BINS, _DIM), lambda i: (0, 0)),
        ],
        out_specs=pl.BlockSpec((_BLK, _NUM_OUTPUTS), lambda i: (i, 0)),
        out_shape=jax.ShapeDtypeStruct((_N, _NUM_OUTPUTS), jnp.float32),
    )(x0g, x1g, tiles)


# speedup vs baseline: 15.4314x; 1.0405x over previous
import jax
import jax.numpy as jnp
from jax.experimental import pallas as pl

_N = 131072
_DIM = 2
_BINS = 15
_BP = _BINS + 1
_NUM_OUTPUTS = _BP * _BP
_BLK = 16384
_SUB = _BLK // 128


def _tile_code_block(x0_ref, x1_ref, tiles_ref, out_ref):
    x0 = x0_ref[...]
    x1 = x1_ref[...]
    cnt0 = jnp.zeros(x0.shape, jnp.int32)
    cnt1 = jnp.zeros(x1.shape, jnp.int32)
    for b in range(_BINS):
        cnt0 = cnt0 + (x0 > tiles_ref[b : b + 1, 0:1]).astype(jnp.int32)
        cnt1 = cnt1 + (x1 > tiles_ref[b : b + 1, 1:2]).astype(jnp.int32)
    code = _BP * cnt0 + cnt1
    cols = jax.lax.broadcasted_iota(jnp.int32, (_SUB, _NUM_OUTPUTS), 1)
    for k in range(128):
        col = code[:, k : k + 1]
        out_ref[_SUB * k : _SUB * (k + 1), :] = (cols == col).astype(
            jnp.float32
        )


def kernel(x, tiles):
    nb = _N // _BLK
    x0g = x[:, 0].reshape(nb, 128, _SUB).transpose(0, 2, 1).reshape(nb * _SUB, 128)
    x1g = x[:, 1].reshape(nb, 128, _SUB).transpose(0, 2, 1).reshape(nb * _SUB, 128)
    return pl.pallas_call(
        _tile_code_block,
        grid=(nb,),
        in_specs=[
            pl.BlockSpec((_SUB, 128), lambda i: (i, 0)),
            pl.BlockSpec((_SUB, 128), lambda i: (i, 0)),
            pl.BlockSpec((_BINS, _DIM), lambda i: (0, 0)),
        ],
        out_specs=pl.BlockSpec((_BLK, _NUM_OUTPUTS), lambda i: (i, 0)),
        out_shape=jax.ShapeDtypeStruct((_N, _NUM_OUTPUTS), jnp.float32),
    )(x0g, x1g, tiles)
